# baseline (speedup 1.0000x reference)
import functools

import jax
import jax.numpy as jnp
from jax import lax
from jax.experimental import pallas as pl
from jax.experimental.pallas import tpu as pltpu
from jax.experimental.pallas import tpu_sc as plsc

_VOCAB = 100000
_EMB = 128
_BATCH = 16384
_NC = 2
_NS = 16
_NW = _NC * _NS
_BPW = _BATCH // _NW
_C = 128
_NCHUNK = _BPW // _C
_L = 16


def _dot_chunk(r1, r2, out_v, out_base):
    m_last = lax.iota(jnp.int32, _L) == (_L - 1)

    @plsc.parallel_loop(0, _C // (2 * _L), unroll=1)
    def _body(g):
        for j in range(2 * _L):
            r = g * (2 * _L) + j
            p = [r1[r, pl.ds(k * _L, _L)] * r2[r, pl.ds(k * _L, _L)]
                 for k in range(_EMB // _L)]
            a = ((p[0] + p[1]) + (p[2] + p[3])) + ((p[4] + p[5]) + (p[6] + p[7]))
            cs = plsc.cumsum(a)
            plsc.store_compressed(out_v.at[pl.ds(out_base + r, _L)], cs,
                                  mask=m_last)


def _sigmoid_all(out_v):
    @plsc.parallel_loop(0, _BPW // _L, unroll=4)
    def _body(g):
        v = out_v[pl.ds(g * _L, _L)]
        out_v[pl.ds(g * _L, _L)] = 1.0 / (1.0 + jnp.exp(-v))


_mesh = plsc.VectorSubcoreMesh(core_axis_name="c", subcore_axis_name="s")


@functools.partial(
    pl.kernel,
    out_type=jax.ShapeDtypeStruct((_BATCH,), jnp.float32),
    mesh=_mesh,
    scratch_types=[
        pltpu.VMEM((_BPW,), jnp.int32),
        pltpu.VMEM((_BPW,), jnp.int32),
        pltpu.VMEM((_C, _EMB), jnp.float32),
        pltpu.VMEM((_C, _EMB), jnp.float32),
        pltpu.VMEM((_C, _EMB), jnp.float32),
        pltpu.VMEM((_C, _EMB), jnp.float32),
        pltpu.VMEM((_BPW + _L,), jnp.float32),
        pltpu.SemaphoreType.DMA,
        pltpu.SemaphoreType.DMA,
    ],
    compiler_params=pltpu.CompilerParams(needs_layout_passes=False),
)
def _contrastive_sc(w1_hbm, w2_hbm, e1_hbm, e2_hbm, out_hbm,
                    idx1_v, idx2_v, r1a, r1b, r2a, r2b, out_v, sem0, sem1):
    wid = lax.axis_index("s") * _NC + lax.axis_index("c")
    base = wid * _BPW

    icp1 = pltpu.async_copy(w1_hbm.at[pl.ds(base, _BPW)], idx1_v, sem0)
    icp2 = pltpu.async_copy(w2_hbm.at[pl.ds(base, _BPW)], idx2_v, sem1)
    icp1.wait()
    icp2.wait()

    r1 = (r1a, r1b)
    r2 = (r2a, r2b)
    sems = (sem0, sem1)

    def start(c):
        slot = c % 2
        cp1 = pltpu.async_copy(
            e1_hbm.at[idx1_v.at[pl.ds(c * _C, _C)]], r1[slot], sems[slot])
        cp2 = pltpu.async_copy(
            e2_hbm.at[idx2_v.at[pl.ds(c * _C, _C)]], r2[slot], sems[slot])
        return cp1, cp2

    cps = start(0)
    for c in range(_NCHUNK):
        cur = cps
        if c + 1 < _NCHUNK:
            cps = start(c + 1)
        cur[0].wait()
        cur[1].wait()
        slot = c % 2
        _dot_chunk(r1[slot], r2[slot], out_v, c * _C)

    _sigmoid_all(out_v)
    pltpu.sync_copy(out_v.at[pl.ds(0, _BPW)], out_hbm.at[pl.ds(base, _BPW)])


def kernel(word1, word2, E1, E2):
    return _contrastive_sc(word1.astype(jnp.int32), word2.astype(jnp.int32),
                           E1, E2)


# speedup vs baseline: 1.2471x; 1.2471x over previous
import functools

import jax
import jax.numpy as jnp
from jax import lax
from jax.experimental import pallas as pl
from jax.experimental.pallas import tpu as pltpu
from jax.experimental.pallas import tpu_sc as plsc

_VOCAB = 100000
_EMB = 128
_BATCH = 16384
_NC = 2
_NS = 16
_NW = _NC * _NS
_BPW = _BATCH // _NW
_C = 128
_NCHUNK = _BPW // _C
_L = 16


def _dot_chunk(r1, r2, out_v, out_base):
    m_last = lax.iota(jnp.int32, _L) == (_L - 1)

    @plsc.parallel_loop(0, _C // 8, unroll=1)
    def _body(g):
        for j in range(8):
            r = g * 8 + j
            p = [r1[r, pl.ds(k * _L, _L)] * r2[r, pl.ds(k * _L, _L)]
                 for k in range(_EMB // _L)]
            a = ((p[0] + p[1]) + (p[2] + p[3])) + ((p[4] + p[5]) + (p[6] + p[7]))
            cs = plsc.cumsum(a)
            plsc.store_compressed(out_v.at[pl.ds(out_base + r, _L)], cs,
                                  mask=m_last)


def _sigmoid_all(out_v):
    @plsc.parallel_loop(0, _BPW // _L, unroll=4)
    def _body(g):
        v = out_v[pl.ds(g * _L, _L)]
        out_v[pl.ds(g * _L, _L)] = 1.0 / (1.0 + jnp.exp(-v))


_mesh = plsc.VectorSubcoreMesh(core_axis_name="c", subcore_axis_name="s")


@functools.partial(
    pl.kernel,
    out_type=jax.ShapeDtypeStruct((_BATCH,), jnp.float32),
    mesh=_mesh,
    scratch_types=[
        pltpu.VMEM((_BPW,), jnp.int32),
        pltpu.VMEM((_BPW,), jnp.int32),
        pltpu.VMEM((_C, _EMB), jnp.float32),
        pltpu.VMEM((_C, _EMB), jnp.float32),
        pltpu.VMEM((_C, _EMB), jnp.float32),
        pltpu.VMEM((_C, _EMB), jnp.float32),
        pltpu.VMEM((_BPW + _L,), jnp.float32),
        pltpu.SemaphoreType.DMA,
        pltpu.SemaphoreType.DMA,
    ],
    compiler_params=pltpu.CompilerParams(needs_layout_passes=False),
)
def _contrastive_sc(w1_hbm, w2_hbm, e1_hbm, e2_hbm, out_hbm,
                    idx1_v, idx2_v, r1a, r1b, r2a, r2b, out_v, sem0, sem1):
    wid = lax.axis_index("s") * _NC + lax.axis_index("c")
    base = wid * _BPW

    icp1 = pltpu.async_copy(w1_hbm.at[pl.ds(base, _BPW)], idx1_v, sem0)
    icp2 = pltpu.async_copy(w2_hbm.at[pl.ds(base, _BPW)], idx2_v, sem1)
    icp1.wait()
    icp2.wait()

    r1 = (r1a, r1b)
    r2 = (r2a, r2b)
    sems = (sem0, sem1)

    def start(c):
        slot = c % 2
        cp1 = pltpu.async_copy(
            e1_hbm.at[idx1_v.at[pl.ds(c * _C, _C)]], r1[slot], sems[slot])
        cp2 = pltpu.async_copy(
            e2_hbm.at[idx2_v.at[pl.ds(c * _C, _C)]], r2[slot], sems[slot])
        return cp1, cp2

    cps = start(0)
    for c in range(_NCHUNK):
        cur = cps
        if c + 1 < _NCHUNK:
            cps = start(c + 1)
        cur[0].wait()
        cur[1].wait()
        slot = c % 2
        _dot_chunk(r1[slot], r2[slot], out_v, c * _C)

    _sigmoid_all(out_v)
    pltpu.sync_copy(out_v.at[pl.ds(0, _BPW)], out_hbm.at[pl.ds(base, _BPW)])


def kernel(word1, word2, E1, E2):
    return _contrastive_sc(word1.astype(jnp.int32), word2.astype(jnp.int32),
                           E1, E2)


# speedup vs baseline: 1.3516x; 1.0839x over previous
import functools

import jax
import jax.numpy as jnp
from jax import lax
from jax.experimental import pallas as pl
from jax.experimental.pallas import tpu as pltpu
from jax.experimental.pallas import tpu_sc as plsc

_VOCAB = 100000
_EMB = 128
_BATCH = 16384
_NC = 2
_NS = 16
_NW = _NC * _NS
_BPW = _BATCH // _NW
_C = 128
_NCHUNK = _BPW // _C
_L = 16


def _dot_chunk(r1, r2, out_v, out_base):
    m_last = lax.iota(jnp.int32, _L) == (_L - 1)

    @plsc.parallel_loop(0, _C // 4, unroll=1)
    def _body(g):
        for j in range(4):
            r = g * 4 + j
            p = [r1[r, pl.ds(k * _L, _L)] * r2[r, pl.ds(k * _L, _L)]
                 for k in range(_EMB // _L)]
            a = ((p[0] + p[1]) + (p[2] + p[3])) + ((p[4] + p[5]) + (p[6] + p[7]))
            cs = plsc.cumsum(a)
            plsc.store_compressed(out_v.at[pl.ds(out_base + r, _L)], cs,
                                  mask=m_last)


def _sigmoid_all(out_v):
    @plsc.parallel_loop(0, _BPW // _L, unroll=4)
    def _body(g):
        v = out_v[pl.ds(g * _L, _L)]
        out_v[pl.ds(g * _L, _L)] = 1.0 / (1.0 + jnp.exp(-v))


_mesh = plsc.VectorSubcoreMesh(core_axis_name="c", subcore_axis_name="s")


@functools.partial(
    pl.kernel,
    out_type=jax.ShapeDtypeStruct((_BATCH,), jnp.float32),
    mesh=_mesh,
    scratch_types=[
        pltpu.VMEM((_BPW,), jnp.int32),
        pltpu.VMEM((_BPW,), jnp.int32),
        pltpu.VMEM((_C, _EMB), jnp.float32),
        pltpu.VMEM((_C, _EMB), jnp.float32),
        pltpu.VMEM((_C, _EMB), jnp.float32),
        pltpu.VMEM((_C, _EMB), jnp.float32),
        pltpu.VMEM((_BPW + _L,), jnp.float32),
        pltpu.SemaphoreType.DMA,
        pltpu.SemaphoreType.DMA,
    ],
    compiler_params=pltpu.CompilerParams(needs_layout_passes=False),
)
def _contrastive_sc(w1_hbm, w2_hbm, e1_hbm, e2_hbm, out_hbm,
                    idx1_v, idx2_v, r1a, r1b, r2a, r2b, out_v, sem0, sem1):
    wid = lax.axis_index("s") * _NC + lax.axis_index("c")
    base = wid * _BPW

    icp1 = pltpu.async_copy(w1_hbm.at[pl.ds(base, _BPW)], idx1_v, sem0)
    icp2 = pltpu.async_copy(w2_hbm.at[pl.ds(base, _BPW)], idx2_v, sem1)
    icp1.wait()
    icp2.wait()

    r1 = (r1a, r1b)
    r2 = (r2a, r2b)
    sems = (sem0, sem1)

    def start(c):
        slot = c % 2
        cp1 = pltpu.async_copy(
            e1_hbm.at[idx1_v.at[pl.ds(c * _C, _C)]], r1[slot], sems[slot])
        cp2 = pltpu.async_copy(
            e2_hbm.at[idx2_v.at[pl.ds(c * _C, _C)]], r2[slot], sems[slot])
        return cp1, cp2

    cps = start(0)
    for c in range(_NCHUNK):
        cur = cps
        if c + 1 < _NCHUNK:
            cps = start(c + 1)
        cur[0].wait()
        cur[1].wait()
        slot = c % 2
        _dot_chunk(r1[slot], r2[slot], out_v, c * _C)

    _sigmoid_all(out_v)
    pltpu.sync_copy(out_v.at[pl.ds(0, _BPW)], out_hbm.at[pl.ds(base, _BPW)])


def kernel(word1, word2, E1, E2):
    return _contrastive_sc(word1.astype(jnp.int32), word2.astype(jnp.int32),
                           E1, E2)


# speedup vs baseline: 1.4159x; 1.0476x over previous
import functools

import jax
import jax.numpy as jnp
from jax import lax
from jax.experimental import pallas as pl
from jax.experimental.pallas import tpu as pltpu
from jax.experimental.pallas import tpu_sc as plsc

_VOCAB = 100000
_EMB = 128
_BATCH = 16384
_NC = 2
_NS = 16
_NW = _NC * _NS
_BPW = _BATCH // _NW
_C = 128
_NCHUNK = _BPW // _C
_L = 16


def _dot_chunk(r1, r2, out_v, out_base):
    m_last = lax.iota(jnp.int32, _L) == (_L - 1)

    @plsc.parallel_loop(0, _C // 2, unroll=1)
    def _body(g):
        for j in range(2):
            r = g * 2 + j
            p = [r1[r, pl.ds(k * _L, _L)] * r2[r, pl.ds(k * _L, _L)]
                 for k in range(_EMB // _L)]
            a = ((p[0] + p[1]) + (p[2] + p[3])) + ((p[4] + p[5]) + (p[6] + p[7]))
            cs = plsc.cumsum(a)
            plsc.store_compressed(out_v.at[pl.ds(out_base + r, _L)], cs,
                                  mask=m_last)


def _sigmoid_all(out_v):
    @plsc.parallel_loop(0, _BPW // _L, unroll=4)
    def _body(g):
        v = out_v[pl.ds(g * _L, _L)]
        out_v[pl.ds(g * _L, _L)] = 1.0 / (1.0 + jnp.exp(-v))


_mesh = plsc.VectorSubcoreMesh(core_axis_name="c", subcore_axis_name="s")


@functools.partial(
    pl.kernel,
    out_type=jax.ShapeDtypeStruct((_BATCH,), jnp.float32),
    mesh=_mesh,
    scratch_types=[
        pltpu.VMEM((_BPW,), jnp.int32),
        pltpu.VMEM((_BPW,), jnp.int32),
        pltpu.VMEM((_C, _EMB), jnp.float32),
        pltpu.VMEM((_C, _EMB), jnp.float32),
        pltpu.VMEM((_C, _EMB), jnp.float32),
        pltpu.VMEM((_C, _EMB), jnp.float32),
        pltpu.VMEM((_BPW + _L,), jnp.float32),
        pltpu.SemaphoreType.DMA,
        pltpu.SemaphoreType.DMA,
    ],
    compiler_params=pltpu.CompilerParams(needs_layout_passes=False),
)
def _contrastive_sc(w1_hbm, w2_hbm, e1_hbm, e2_hbm, out_hbm,
                    idx1_v, idx2_v, r1a, r1b, r2a, r2b, out_v, sem0, sem1):
    wid = lax.axis_index("s") * _NC + lax.axis_index("c")
    base = wid * _BPW

    icp1 = pltpu.async_copy(w1_hbm.at[pl.ds(base, _BPW)], idx1_v, sem0)
    icp2 = pltpu.async_copy(w2_hbm.at[pl.ds(base, _BPW)], idx2_v, sem1)
    icp1.wait()
    icp2.wait()

    r1 = (r1a, r1b)
    r2 = (r2a, r2b)
    sems = (sem0, sem1)

    def start(c):
        slot = c % 2
        cp1 = pltpu.async_copy(
            e1_hbm.at[idx1_v.at[pl.ds(c * _C, _C)]], r1[slot], sems[slot])
        cp2 = pltpu.async_copy(
            e2_hbm.at[idx2_v.at[pl.ds(c * _C, _C)]], r2[slot], sems[slot])
        return cp1, cp2

    cps = start(0)
    for c in range(_NCHUNK):
        cur = cps
        if c + 1 < _NCHUNK:
            cps = start(c + 1)
        cur[0].wait()
        cur[1].wait()
        slot = c % 2
        _dot_chunk(r1[slot], r2[slot], out_v, c * _C)

    _sigmoid_all(out_v)
    pltpu.sync_copy(out_v.at[pl.ds(0, _BPW)], out_hbm.at[pl.ds(base, _BPW)])


def kernel(word1, word2, E1, E2):
    return _contrastive_sc(word1.astype(jnp.int32), word2.astype(jnp.int32),
                           E1, E2)


# speedup vs baseline: 1.4364x; 1.0145x over previous
import functools

import jax
import jax.numpy as jnp
from jax import lax
from jax.experimental import pallas as pl
from jax.experimental.pallas import tpu as pltpu
from jax.experimental.pallas import tpu_sc as plsc

_VOCAB = 100000
_EMB = 128
_BATCH = 16384
_NC = 2
_NS = 16
_NW = _NC * _NS
_BPW = _BATCH // _NW
_C = 128
_NCHUNK = _BPW // _C
_L = 16


def _dot_chunk(r1, r2, out_v, out_base):
    m_last = lax.iota(jnp.int32, _L) == (_L - 1)

    @plsc.parallel_loop(0, _C, unroll=1)
    def _body(g):
        for j in range(1):
            r = g + j
            p = [r1[r, pl.ds(k * _L, _L)] * r2[r, pl.ds(k * _L, _L)]
                 for k in range(_EMB // _L)]
            a = ((p[0] + p[1]) + (p[2] + p[3])) + ((p[4] + p[5]) + (p[6] + p[7]))
            cs = plsc.cumsum(a)
            plsc.store_compressed(out_v.at[pl.ds(out_base + r, _L)], cs,
                                  mask=m_last)


def _sigmoid_all(out_v):
    @plsc.parallel_loop(0, _BPW // _L, unroll=4)
    def _body(g):
        v = out_v[pl.ds(g * _L, _L)]
        out_v[pl.ds(g * _L, _L)] = 1.0 / (1.0 + jnp.exp(-v))


_mesh = plsc.VectorSubcoreMesh(core_axis_name="c", subcore_axis_name="s")


@functools.partial(
    pl.kernel,
    out_type=jax.ShapeDtypeStruct((_BATCH,), jnp.float32),
    mesh=_mesh,
    scratch_types=[
        pltpu.VMEM((_BPW,), jnp.int32),
        pltpu.VMEM((_BPW,), jnp.int32),
        pltpu.VMEM((_C, _EMB), jnp.float32),
        pltpu.VMEM((_C, _EMB), jnp.float32),
        pltpu.VMEM((_C, _EMB), jnp.float32),
        pltpu.VMEM((_C, _EMB), jnp.float32),
        pltpu.VMEM((_BPW + _L,), jnp.float32),
        pltpu.SemaphoreType.DMA,
        pltpu.SemaphoreType.DMA,
    ],
    compiler_params=pltpu.CompilerParams(needs_layout_passes=False),
)
def _contrastive_sc(w1_hbm, w2_hbm, e1_hbm, e2_hbm, out_hbm,
                    idx1_v, idx2_v, r1a, r1b, r2a, r2b, out_v, sem0, sem1):
    wid = lax.axis_index("s") * _NC + lax.axis_index("c")
    base = wid * _BPW

    icp1 = pltpu.async_copy(w1_hbm.at[pl.ds(base, _BPW)], idx1_v, sem0)
    icp2 = pltpu.async_copy(w2_hbm.at[pl.ds(base, _BPW)], idx2_v, sem1)
    icp1.wait()
    icp2.wait()

    r1 = (r1a, r1b)
    r2 = (r2a, r2b)
    sems = (sem0, sem1)

    def start(c):
        slot = c % 2
        cp1 = pltpu.async_copy(
            e1_hbm.at[idx1_v.at[pl.ds(c * _C, _C)]], r1[slot], sems[slot])
        cp2 = pltpu.async_copy(
            e2_hbm.at[idx2_v.at[pl.ds(c * _C, _C)]], r2[slot], sems[slot])
        return cp1, cp2

    cps = start(0)
    for c in range(_NCHUNK):
        cur = cps
        if c + 1 < _NCHUNK:
            cps = start(c + 1)
        cur[0].wait()
        cur[1].wait()
        slot = c % 2
        _dot_chunk(r1[slot], r2[slot], out_v, c * _C)

    _sigmoid_all(out_v)
    pltpu.sync_copy(out_v.at[pl.ds(0, _BPW)], out_hbm.at[pl.ds(base, _BPW)])


def kernel(word1, word2, E1, E2):
    return _contrastive_sc(word1.astype(jnp.int32), word2.astype(jnp.int32),
                           E1, E2)
